# baseline (speedup 1.0000x reference)
import functools

import jax
import jax.numpy as jnp
from jax import lax
from jax.experimental import pallas as pl
from jax.experimental.pallas import tpu as pltpu
from jax.experimental.pallas import tpu_sc as plsc

_B, _S = 16384, 50
_D = 32
_NW = 32
_CPU = 4
_UN = _CPU * 128


def _transpose_unit(g, st):
    d16 = lax.broadcasted_iota(jnp.int32, (16,), 0)
    i1_lo = d16 // 8
    i1_hi = i1_lo + 2
    i2 = d16 % 8

    def tj(jo, carry):
        cb = jo // 16
        k8 = jo % 16
        row0 = cb * 128 + k8 * 8
        cbv = jnp.full((16,), cb, jnp.int32)
        for u in range(8):
            j = row0 + u
            nnv = jnp.full((16,), k8 * 8 + u, jnp.int32)
            vlo = g[j, pl.ds(0, 16)]
            vhi = g[j, pl.ds(16, 16)]
            plsc.store_scatter(st, [cbv, i1_lo, i2, nnv], vlo)
            plsc.store_scatter(st, [cbv, i1_hi, i2, nnv], vhi)
        return carry

    lax.fori_loop(0, _CPU * 16, tj, 0)


def _body(tok_hbm, table_hbm, out_hbm, tokv, idx_a, idx_b, g_a, g_b,
          st_a, st_b, gs_a, gs_b, ss_a, ss_b):
    cid = lax.axis_index("c")
    sid = lax.axis_index("s")
    wid = sid * 2 + cid
    c0 = wid * _CPU
    pltpu.sync_copy(tok_hbm.at[pl.ds(wid * _UN, _UN)],
                    tokv.at[:, pl.ds(0, 64)])
    iota = lax.broadcasted_iota(jnp.int32, (16,), 0)

    def fire(s, idx_v, g, gsem):
        sv = jnp.full((16,), s, jnp.int32)
        for j in range(_UN // 16):
            rows = j * 16 + iota
            idx_v[pl.ds(j * 16, 16)] = plsc.load_gather(tokv, [rows, sv])
        pltpu.async_copy(table_hbm.at[idx_v], g, gsem)

    def drain(idx_v, g, gsem):
        pltpu.make_async_copy(table_hbm.at[idx_v], g, gsem).wait()

    def store(s, st, ssem):
        for c in range(_CPU):
            for r in range(4):
                pltpu.async_copy(st.at[c, r, :, pl.ds(0, 128)],
                                 out_hbm.at[s, r, c0 + c], ssem)

    def wait_store(st, ssem):
        for c in range(_CPU):
            for r in range(4):
                pltpu.make_async_copy(st.at[c, r, :, pl.ds(0, 128)],
                                      out_hbm.at[0, r, 0], ssem).wait()

    fire(0, idx_a, g_a, gs_a)

    def step(p, carry):
        s0 = 2 * p
        fire(s0 + 1, idx_b, g_b, gs_b)
        drain(idx_a, g_a, gs_a)
        @pl.when(p > 0)
        def _():
            wait_store(st_a, ss_a)
        _transpose_unit(g_a, st_a)
        store(s0, st_a, ss_a)
        @pl.when(p < _S // 2 - 1)
        def _():
            fire(s0 + 2, idx_a, g_a, gs_a)
        drain(idx_b, g_b, gs_b)
        @pl.when(p > 0)
        def _():
            wait_store(st_b, ss_b)
        _transpose_unit(g_b, st_b)
        store(s0 + 1, st_b, ss_b)
        return carry

    lax.fori_loop(0, _S // 2, step, 0)
    wait_store(st_a, ss_a)
    wait_store(st_b, ss_b)


@jax.jit
def _embed(tok2, weight):
    mesh = plsc.VectorSubcoreMesh(core_axis_name="c", subcore_axis_name="s")
    kern = functools.partial(
        pl.kernel,
        mesh=mesh,
        out_type=jax.ShapeDtypeStruct((_S, 4, 128, 8, 128), jnp.float32),
        scratch_types=[
            pltpu.VMEM((_UN, 65), jnp.int32),
            pltpu.VMEM((_UN,), jnp.int32),
            pltpu.VMEM((_UN,), jnp.int32),
            pltpu.VMEM((_UN, _D), jnp.float32),
            pltpu.VMEM((_UN, _D), jnp.float32),
            pltpu.VMEM((_CPU, 4, 8, 133), jnp.float32),
            pltpu.VMEM((_CPU, 4, 8, 133), jnp.float32),
            pltpu.SemaphoreType.DMA,
            pltpu.SemaphoreType.DMA,
            pltpu.SemaphoreType.DMA,
            pltpu.SemaphoreType.DMA,
        ],
        compiler_params=pltpu.CompilerParams(
            use_tc_tiling_on_sc=False, needs_layout_passes=False),
    )(_body)
    return kern(tok2, weight)


def kernel(token_ids, weight):
    tokp = jnp.pad(token_ids.astype(jnp.int32), ((0, 0), (0, 64 - _S)))
    out5 = _embed(tokp, weight)
    return out5.transpose((2, 4, 0, 1, 3)).reshape(_B, _S, _D)


# speedup vs baseline: 1.0125x; 1.0125x over previous
import functools

import jax
import jax.numpy as jnp
from jax import lax
from jax.experimental import pallas as pl
from jax.experimental.pallas import tpu as pltpu
from jax.experimental.pallas import tpu_sc as plsc

_B, _S = 16384, 50
_D = 32
_NW = 32
_CPU = 4
_UN = _CPU * 128


def _transpose_unit(g, st):
    d16 = lax.broadcasted_iota(jnp.int32, (16,), 0)
    i1_lo = d16 // 8
    i1_hi = i1_lo + 2
    i2 = d16 % 8

    def tj(jo, carry):
        cb = jo // 16
        k8 = jo % 16
        row0 = cb * 128 + k8 * 8
        cbv = jnp.full((16,), cb, jnp.int32)
        for u in range(8):
            j = row0 + u
            nnv = jnp.full((16,), k8 * 8 + u, jnp.int32)
            vlo = g[j, pl.ds(0, 16)]
            vhi = g[j, pl.ds(16, 16)]
            plsc.store_scatter(st, [cbv, i1_lo, i2, nnv], vlo)
            plsc.store_scatter(st, [cbv, i1_hi, i2, nnv], vhi)
        return carry

    lax.fori_loop(0, _CPU * 16, tj, 0)


def _body(tok_hbm, table_hbm, out_hbm, tokv, idx_a, idx_b, g_a, g_b,
          st_a, st_b, gs_a, gs_b, ss_a, ss_b):
    cid = lax.axis_index("c")
    sid = lax.axis_index("s")
    wid = sid * 2 + cid
    c0 = wid * _CPU
    pltpu.sync_copy(tok_hbm.at[pl.ds(wid * (_UN * _S // 128), _UN * _S // 128)],
                    tokv)
    iota50 = lax.broadcasted_iota(jnp.int32, (16,), 0) * _S

    def fire(s, idx_v, g, gsem):
        for j in range(_UN // 16):
            off = iota50 + (j * 16 * _S + s)
            idx_v[pl.ds(j * 16, 16)] = plsc.load_gather(
                tokv, [off >> 7, off & 127])
        pltpu.async_copy(table_hbm.at[idx_v], g, gsem)

    def drain(idx_v, g, gsem):
        pltpu.make_async_copy(table_hbm.at[idx_v], g, gsem).wait()

    def store(s, st, ssem):
        for c in range(_CPU):
            for r in range(4):
                pltpu.async_copy(st.at[c, r, :, pl.ds(0, 128)],
                                 out_hbm.at[s, r, c0 + c], ssem)

    def wait_store(st, ssem):
        for c in range(_CPU):
            for r in range(4):
                pltpu.make_async_copy(st.at[c, r, :, pl.ds(0, 128)],
                                      out_hbm.at[0, r, 0], ssem).wait()

    fire(0, idx_a, g_a, gs_a)

    def step(p, carry):
        s0 = 2 * p
        fire(s0 + 1, idx_b, g_b, gs_b)
        drain(idx_a, g_a, gs_a)
        @pl.when(p > 0)
        def _():
            wait_store(st_a, ss_a)
        _transpose_unit(g_a, st_a)
        store(s0, st_a, ss_a)
        @pl.when(p < _S // 2 - 1)
        def _():
            fire(s0 + 2, idx_a, g_a, gs_a)
        drain(idx_b, g_b, gs_b)
        @pl.when(p > 0)
        def _():
            wait_store(st_b, ss_b)
        _transpose_unit(g_b, st_b)
        store(s0 + 1, st_b, ss_b)
        return carry

    lax.fori_loop(0, _S // 2, step, 0)
    wait_store(st_a, ss_a)
    wait_store(st_b, ss_b)


@jax.jit
def _embed(tok2, weight):
    mesh = plsc.VectorSubcoreMesh(core_axis_name="c", subcore_axis_name="s")
    kern = functools.partial(
        pl.kernel,
        mesh=mesh,
        out_type=jax.ShapeDtypeStruct((_S, 4, 128, 8, 128), jnp.float32),
        scratch_types=[
            pltpu.VMEM((_UN * _S // 128, 128), jnp.int32),
            pltpu.VMEM((_UN,), jnp.int32),
            pltpu.VMEM((_UN,), jnp.int32),
            pltpu.VMEM((_UN, _D), jnp.float32),
            pltpu.VMEM((_UN, _D), jnp.float32),
            pltpu.VMEM((_CPU, 4, 8, 133), jnp.float32),
            pltpu.VMEM((_CPU, 4, 8, 133), jnp.float32),
            pltpu.SemaphoreType.DMA,
            pltpu.SemaphoreType.DMA,
            pltpu.SemaphoreType.DMA,
            pltpu.SemaphoreType.DMA,
        ],
        compiler_params=pltpu.CompilerParams(
            use_tc_tiling_on_sc=False, needs_layout_passes=False),
    )(_body)
    return kern(tok2, weight)


def kernel(token_ids, weight):
    tok2 = token_ids.reshape(_B * _S // 128, 128).astype(jnp.int32)
    out5 = _embed(tok2, weight)
    return out5.transpose((2, 4, 0, 1, 3)).reshape(_B, _S, _D)
